# baseline (speedup 1.0000x reference)
import functools

import jax
import jax.numpy as jnp
from jax import lax
from jax.experimental import pallas as pl
from jax.experimental.pallas import tpu as pltpu
from jax.experimental.pallas import tpu_sc as plsc

NC = 2
NS = 16
LANES = 16
NW = NC * NS
CHUNK = 128
HIGHEST = lax.Precision.HIGHEST


def _sc_mesh():
    return plsc.VectorSubcoreMesh(core_axis_name="c", subcore_axis_name="s")


def _fill_vmem(ref, rows, cols, value):
    val = jnp.full((LANES,), value, jnp.float32)

    @pl.loop(0, rows)
    def _(i):
        @pl.loop(0, cols, step=LANES)
        def _(j):
            ref[i, pl.ds(j, LANES)] = val


SF = 6
SS = 2


def _edge_pass(table, src3, dst3, NP, C):
    SEG = src3.shape[1]
    rows_per_sub = NP // NS
    spread = rows_per_sub // CHUNK

    @functools.partial(
        pl.kernel,
        out_type=jax.ShapeDtypeStruct((NC, NP, C), jnp.float32),
        mesh=_sc_mesh(),
        scratch_types=[
            pltpu.VMEM((SEG, CHUNK), jnp.int32),
            pltpu.VMEM((SEG, CHUNK), jnp.int32),
            pltpu.VMEM((CHUNK, C), jnp.float32),
            pltpu.VMEM((CHUNK, C), jnp.float32),
            pltpu.VMEM_SHARED((NP, C), jnp.float32),
            pltpu.SemaphoreType.DMA,
        ],
    )
    def edge_k(table_hbm, src_hbm, dst_hbm, out_hbm, src_v, dst_v,
               msg0, msg1, acc_sh, gsem):
        c = lax.axis_index("c")
        s = lax.axis_index("s")
        nseg = jnp.where(c == 0, SF, SS)
        base = jnp.where(c == 0, s * SF, NS * SF + s * SS)

        with jax.named_scope("zero_fill"):
            _fill_vmem(msg0, CHUNK, C, 0.0)

        if True:
            with jax.named_scope("zero_spread"):
                @pl.loop(0, spread)
                def _(j):
                    pltpu.sync_copy(
                        msg0,
                        acc_sh.at[pl.ds(s * rows_per_sub + j * CHUNK, CHUNK)])
            with jax.named_scope("barrier0"):
                plsc.subcore_barrier()

            @pl.loop(0, nseg)
            def _(q):
                with jax.named_scope("idx_load"):
                    pltpu.sync_copy(src_hbm.at[base + q], src_v)
                    pltpu.sync_copy(dst_hbm.at[base + q], dst_v)

                pltpu.async_copy(table_hbm.at[src_v.at[0]], msg0, gsem)

                @pl.loop(0, SEG, step=2)
                @jax.named_scope("chunk_pair")
                def _(g):
                    pltpu.make_async_copy(
                        table_hbm.at[src_v.at[g]], msg0, gsem).wait()
                    pltpu.async_copy(table_hbm.at[src_v.at[g + 1]], msg1, gsem)
                    pltpu.sync_copy(msg0, acc_sh.at[dst_v.at[g]], add=True)
                    pltpu.make_async_copy(
                        table_hbm.at[src_v.at[g + 1]], msg1, gsem).wait()

                    @pl.when(g + 2 < SEG)
                    def _():
                        pltpu.async_copy(
                            table_hbm.at[src_v.at[g + 2]], msg0, gsem)

                    pltpu.sync_copy(msg1, acc_sh.at[dst_v.at[g + 1]], add=True)

            with jax.named_scope("barrier1"):
                plsc.subcore_barrier()
            with jax.named_scope("writeout"):
                pltpu.sync_copy(
                    acc_sh.at[pl.ds(s * rows_per_sub, rows_per_sub)],
                    out_hbm.at[c, pl.ds(s * rows_per_sub, rows_per_sub)])

    return edge_k(table, src3, dst3)


def _tc_scale(degp, x_pad, NP, C):
    BR = 256

    def body(dp_ref, x_ref, dis_ref, xs_ref):
        p = dp_ref[0] + dp_ref[1]
        deg = 1.0 + jnp.sum(p, axis=1, keepdims=True) * (1.0 / C)
        dis = lax.rsqrt(deg)
        dis_ref[...] = dis
        xs_ref[...] = x_ref[...] * dis

    return pl.pallas_call(
        body,
        grid=(NP // BR,),
        in_specs=[
            pl.BlockSpec((NC, BR, C), lambda i: (0, i, 0)),
            pl.BlockSpec((BR, C), lambda i: (i, 0)),
        ],
        out_specs=[
            pl.BlockSpec((BR, 1), lambda i: (i, 0)),
            pl.BlockSpec((BR, C), lambda i: (i, 0)),
        ],
        out_shape=[
            jax.ShapeDtypeStruct((NP, 1), jnp.float32),
            jax.ShapeDtypeStruct((NP, C), jnp.float32),
        ],
    )(degp, x_pad)


def _tc_mlp(t1a, t1b, xs, dis, W1, b1r, W2, NP, C, CH):
    BR = 512

    def body(ta, tb, xs_r, dis_r, w1_r, b1_r, w2_r, g_ref):
        d = dis_r[...]
        y1 = d * (ta[...] + tb[...] + xs_r[...])
        h = jnp.dot(y1, w1_r[...], preferred_element_type=jnp.float32,
                    precision=HIGHEST)
        h = jnp.maximum(h + b1_r[...], 0.0)
        g_ref[...] = jnp.dot(d * h, w2_r[...],
                             preferred_element_type=jnp.float32,
                             precision=HIGHEST)

    return pl.pallas_call(
        body,
        grid=(NP // BR,),
        in_specs=[
            pl.BlockSpec((BR, C), lambda i: (i, 0)),
            pl.BlockSpec((BR, C), lambda i: (i, 0)),
            pl.BlockSpec((BR, C), lambda i: (i, 0)),
            pl.BlockSpec((BR, 1), lambda i: (i, 0)),
            pl.BlockSpec((C, CH), lambda i: (0, 0)),
            pl.BlockSpec((1, CH), lambda i: (0, 0)),
            pl.BlockSpec((CH, C), lambda i: (0, 0)),
        ],
        out_specs=pl.BlockSpec((BR, C), lambda i: (i, 0)),
        out_shape=jax.ShapeDtypeStruct((NP, C), jnp.float32),
    )(t1a, t1b, xs, dis, W1, b1r, W2)


def _tc_final(t2a, t2b, g, dis, b2r, NP, C):
    BR = 512

    def body(ta, tb, g_r, dis_r, b2_r, out_ref):
        out_ref[...] = dis_r[...] * (ta[...] + tb[...] + g_r[...]) + b2_r[...]

    return pl.pallas_call(
        body,
        grid=(NP // BR,),
        in_specs=[
            pl.BlockSpec((BR, C), lambda i: (i, 0)),
            pl.BlockSpec((BR, C), lambda i: (i, 0)),
            pl.BlockSpec((BR, C), lambda i: (i, 0)),
            pl.BlockSpec((BR, 1), lambda i: (i, 0)),
            pl.BlockSpec((1, C), lambda i: (0, 0)),
        ],
        out_specs=pl.BlockSpec((BR, C), lambda i: (i, 0)),
        out_shape=jax.ShapeDtypeStruct((NP, C), jnp.float32),
    )(t2a, t2b, g, dis, b2r)


def kernel(x, edge_index, W1, b1, W2, b2):
    N, C = x.shape
    CH = W1.shape[1]
    E = edge_index.shape[1]

    src = edge_index[0].astype(jnp.int32)
    dst = edge_index[1].astype(jnp.int32)

    NP = -(-N // (NS * CHUNK)) * (NS * CHUNK)
    NSEG = NS * (SF + SS)
    SEG = -(-E // (CHUNK * NSEG))
    SEG += SEG % 2
    EP = NSEG * SEG * CHUNK

    src_p = jnp.concatenate(
        [src, jnp.zeros((EP - E,), jnp.int32)]).reshape(NSEG, SEG, CHUNK)
    dst_p = jnp.concatenate(
        [dst, jnp.full((EP - E,), NP - 1, jnp.int32)]).reshape(NSEG, SEG, CHUNK)
    x_pad = jnp.pad(x, ((0, NP - N), (0, 0)))
    b1r = b1.reshape(1, CH)
    b2r = b2.reshape(1, C)

    ones_tab = jnp.ones((NP, C), jnp.float32)
    degp = _edge_pass(ones_tab, src_p, dst_p, NP, C)
    dis, xs = _tc_scale(degp, x_pad, NP, C)
    t1 = _edge_pass(xs, src_p, dst_p, NP, C)
    g = _tc_mlp(t1[0], t1[1], xs, dis, W1, b1r, W2, NP, C, CH)
    t2 = _edge_pass(g, src_p, dst_p, NP, C)
    out = _tc_final(t2[0], t2[1], g, dis, b2r, NP, C)
    return out[:N]


# speedup vs baseline: 2.6154x; 2.6154x over previous
import functools

import jax
import jax.numpy as jnp
from jax import lax
from jax.experimental import pallas as pl
from jax.experimental.pallas import tpu as pltpu
from jax.experimental.pallas import tpu_sc as plsc

NC = 2
NS = 16
LANES = 16
NW = NC * NS
CHUNK = 128
HIGHEST = lax.Precision.HIGHEST


def _sc_mesh():
    return plsc.VectorSubcoreMesh(core_axis_name="c", subcore_axis_name="s")


def _fill_vmem(ref, rows, cols, value):
    val = jnp.full((LANES,), value, jnp.float32)

    @pl.loop(0, rows)
    def _(i):
        @pl.loop(0, cols, step=LANES)
        def _(j):
            ref[i, pl.ds(j, LANES)] = val


SF = 4
SS = 4


def _edge_pass(table, src3, dst3, NP, C):
    SEG = src3.shape[1]
    rows_per_sub = NP // NS
    spread = rows_per_sub // CHUNK

    @functools.partial(
        pl.kernel,
        out_type=jax.ShapeDtypeStruct((NC, NP, C), jnp.float32),
        mesh=_sc_mesh(),
        scratch_types=[
            pltpu.VMEM((SEG, CHUNK), jnp.int32),
            pltpu.VMEM((SEG, CHUNK), jnp.int32),
            pltpu.VMEM((CHUNK, C), jnp.float32),
            pltpu.VMEM((CHUNK, C), jnp.float32),
            pltpu.VMEM_SHARED((NP, C), jnp.float32),
            pltpu.SemaphoreType.DMA,
        ],
    )
    def edge_k(table_hbm, src_hbm, dst_hbm, out_hbm, src_v, dst_v,
               msg0, msg1, acc_sh, gsem):
        c = lax.axis_index("c")
        s = lax.axis_index("s")
        nseg = jnp.where(c == 0, SF, SS)
        base = jnp.where(c == 0, s * SF, NS * SF + s * SS)

        with jax.named_scope("zero_fill"):
            _fill_vmem(msg0, CHUNK, C, 0.0)

        if True:
            with jax.named_scope("zero_spread"):
                @pl.loop(0, spread)
                def _(j):
                    pltpu.sync_copy(
                        msg0,
                        acc_sh.at[pl.ds(s * rows_per_sub + j * CHUNK, CHUNK)])
            with jax.named_scope("barrier0"):
                plsc.subcore_barrier()

            @pl.loop(0, nseg)
            def _(q):
                with jax.named_scope("idx_load"):
                    pltpu.sync_copy(src_hbm.at[base + q], src_v)
                    pltpu.sync_copy(dst_hbm.at[base + q], dst_v)

                pltpu.async_copy(table_hbm.at[src_v.at[0]], msg0, gsem)

                @pl.loop(0, SEG, step=2)
                @jax.named_scope("chunk_pair")
                def _(g):
                    pltpu.make_async_copy(
                        table_hbm.at[src_v.at[g]], msg0, gsem).wait()
                    pltpu.async_copy(table_hbm.at[src_v.at[g + 1]], msg1, gsem)
                    pltpu.sync_copy(msg0, acc_sh.at[dst_v.at[g]], add=True)
                    pltpu.make_async_copy(
                        table_hbm.at[src_v.at[g + 1]], msg1, gsem).wait()

                    @pl.when(g + 2 < SEG)
                    def _():
                        pltpu.async_copy(
                            table_hbm.at[src_v.at[g + 2]], msg0, gsem)

                    pltpu.sync_copy(msg1, acc_sh.at[dst_v.at[g + 1]], add=True)

            with jax.named_scope("barrier1"):
                plsc.subcore_barrier()
            with jax.named_scope("writeout"):
                pltpu.sync_copy(
                    acc_sh.at[pl.ds(s * rows_per_sub, rows_per_sub)],
                    out_hbm.at[c, pl.ds(s * rows_per_sub, rows_per_sub)])

    return edge_k(table, src3, dst3)


def _tc_scale(degp, x_pad, NP, C):
    BR = 256

    def body(dp_ref, x_ref, dis_ref, xs_ref):
        p = dp_ref[0] + dp_ref[1]
        deg = 1.0 + jnp.sum(p, axis=1, keepdims=True) * (1.0 / C)
        dis = lax.rsqrt(deg)
        dis_ref[...] = dis
        xs_ref[...] = x_ref[...] * dis

    return pl.pallas_call(
        body,
        grid=(NP // BR,),
        in_specs=[
            pl.BlockSpec((NC, BR, C), lambda i: (0, i, 0)),
            pl.BlockSpec((BR, C), lambda i: (i, 0)),
        ],
        out_specs=[
            pl.BlockSpec((BR, 1), lambda i: (i, 0)),
            pl.BlockSpec((BR, C), lambda i: (i, 0)),
        ],
        out_shape=[
            jax.ShapeDtypeStruct((NP, 1), jnp.float32),
            jax.ShapeDtypeStruct((NP, C), jnp.float32),
        ],
    )(degp, x_pad)


def _tc_mlp(t1a, t1b, xs, dis, W1, b1r, W2, NP, C, CH):
    BR = 512

    def body(ta, tb, xs_r, dis_r, w1_r, b1_r, w2_r, g_ref):
        d = dis_r[...]
        y1 = d * (ta[...] + tb[...] + xs_r[...])
        h = jnp.dot(y1, w1_r[...], preferred_element_type=jnp.float32,
                    precision=HIGHEST)
        h = jnp.maximum(h + b1_r[...], 0.0)
        g_ref[...] = jnp.dot(d * h, w2_r[...],
                             preferred_element_type=jnp.float32,
                             precision=HIGHEST)

    return pl.pallas_call(
        body,
        grid=(NP // BR,),
        in_specs=[
            pl.BlockSpec((BR, C), lambda i: (i, 0)),
            pl.BlockSpec((BR, C), lambda i: (i, 0)),
            pl.BlockSpec((BR, C), lambda i: (i, 0)),
            pl.BlockSpec((BR, 1), lambda i: (i, 0)),
            pl.BlockSpec((C, CH), lambda i: (0, 0)),
            pl.BlockSpec((1, CH), lambda i: (0, 0)),
            pl.BlockSpec((CH, C), lambda i: (0, 0)),
        ],
        out_specs=pl.BlockSpec((BR, C), lambda i: (i, 0)),
        out_shape=jax.ShapeDtypeStruct((NP, C), jnp.float32),
    )(t1a, t1b, xs, dis, W1, b1r, W2)


def _tc_final(t2a, t2b, g, dis, b2r, NP, C):
    BR = 512

    def body(ta, tb, g_r, dis_r, b2_r, out_ref):
        out_ref[...] = dis_r[...] * (ta[...] + tb[...] + g_r[...]) + b2_r[...]

    return pl.pallas_call(
        body,
        grid=(NP // BR,),
        in_specs=[
            pl.BlockSpec((BR, C), lambda i: (i, 0)),
            pl.BlockSpec((BR, C), lambda i: (i, 0)),
            pl.BlockSpec((BR, C), lambda i: (i, 0)),
            pl.BlockSpec((BR, 1), lambda i: (i, 0)),
            pl.BlockSpec((1, C), lambda i: (0, 0)),
        ],
        out_specs=pl.BlockSpec((BR, C), lambda i: (i, 0)),
        out_shape=jax.ShapeDtypeStruct((NP, C), jnp.float32),
    )(t2a, t2b, g, dis, b2r)


def kernel(x, edge_index, W1, b1, W2, b2):
    N, C = x.shape
    CH = W1.shape[1]
    E = edge_index.shape[1]

    src = edge_index[0].astype(jnp.int32)
    dst = edge_index[1].astype(jnp.int32)

    NP = -(-N // (NS * CHUNK)) * (NS * CHUNK)
    if NP == N:
        NP += NS * CHUNK
    NSEG = NS * (SF + SS)
    SEG = -(-E // (CHUNK * NSEG))
    SEG += SEG % 2
    EP = NSEG * SEG * CHUNK

    pad_i = jnp.arange(EP - E, dtype=jnp.int32)
    src_p = jnp.concatenate(
        [src, pad_i % N]).reshape(NSEG, SEG, CHUNK)
    dst_p = jnp.concatenate(
        [dst, N + pad_i % (NP - N)]).reshape(NSEG, SEG, CHUNK)
    x_pad = jnp.pad(x, ((0, NP - N), (0, 0)))
    b1r = b1.reshape(1, CH)
    b2r = b2.reshape(1, C)

    ones_tab = jnp.ones((NP, C), jnp.float32)
    degp = _edge_pass(ones_tab, src_p, dst_p, NP, C)
    dis, xs = _tc_scale(degp, x_pad, NP, C)
    t1 = _edge_pass(xs, src_p, dst_p, NP, C)
    g = _tc_mlp(t1[0], t1[1], xs, dis, W1, b1r, W2, NP, C, CH)
    t2 = _edge_pass(g, src_p, dst_p, NP, C)
    out = _tc_final(t2[0], t2[1], g, dis, b2r, NP, C)
    return out[:N]


# speedup vs baseline: 2.7418x; 1.0484x over previous
import functools

import jax
import jax.numpy as jnp
from jax import lax
from jax.experimental import pallas as pl
from jax.experimental.pallas import tpu as pltpu
from jax.experimental.pallas import tpu_sc as plsc

NC = 2
NS = 16
LANES = 16
NW = NC * NS
CHUNK = 128
HIGHEST = lax.Precision.HIGHEST


def _sc_mesh():
    return plsc.VectorSubcoreMesh(core_axis_name="c", subcore_axis_name="s")


def _fill_vmem(ref, rows, cols, value):
    val = jnp.full((LANES,), value, jnp.float32)

    @pl.loop(0, rows)
    def _(i):
        @pl.loop(0, cols, step=LANES)
        def _(j):
            ref[i, pl.ds(j, LANES)] = val


SF = 4
SS = 4


def _edge_pass(table, src3, dst3, NP, C):
    SEG = src3.shape[1]
    rows_per_sub = NP // NS
    spread = rows_per_sub // CHUNK

    @functools.partial(
        pl.kernel,
        out_type=jax.ShapeDtypeStruct((NC, NP, C), jnp.float32),
        mesh=_sc_mesh(),
        scratch_types=[
            pltpu.VMEM((SEG, CHUNK), jnp.int32),
            pltpu.VMEM((SEG, CHUNK), jnp.int32),
            pltpu.VMEM((CHUNK, C), jnp.float32),
            pltpu.VMEM((CHUNK, C), jnp.float32),
            pltpu.VMEM_SHARED((NP, C), jnp.float32),
            pltpu.SemaphoreType.DMA,
        ],
    )
    def edge_k(table_hbm, src_hbm, dst_hbm, out_hbm, src_v, dst_v,
               msg0, msg1, acc_sh, gsem):
        c = lax.axis_index("c")
        s = lax.axis_index("s")
        nseg = jnp.where(c == 0, SF, SS)
        base = jnp.where(c == 0, s * SF, NS * SF + s * SS)

        with jax.named_scope("zero_fill"):
            _fill_vmem(msg0, CHUNK, C, 0.0)

        if True:
            with jax.named_scope("zero_spread"):
                @pl.loop(0, spread)
                def _(j):
                    pltpu.sync_copy(
                        msg0,
                        acc_sh.at[pl.ds(s * rows_per_sub + j * CHUNK, CHUNK)])
            with jax.named_scope("barrier0"):
                plsc.subcore_barrier()

            @pl.loop(0, nseg)
            def _(q):
                with jax.named_scope("idx_load"):
                    pltpu.sync_copy(src_hbm.at[base + q], src_v)
                    pltpu.sync_copy(dst_hbm.at[base + q], dst_v)

                pltpu.async_copy(table_hbm.at[src_v.at[0]], msg0, gsem)

                @pl.loop(0, SEG, step=2)
                @jax.named_scope("chunk_pair")
                def _(g):
                    pltpu.make_async_copy(
                        table_hbm.at[src_v.at[g]], msg0, gsem).wait()
                    pltpu.async_copy(table_hbm.at[src_v.at[g + 1]], msg1, gsem)
                    pltpu.sync_copy(msg0, acc_sh.at[dst_v.at[g]], add=True)
                    pltpu.make_async_copy(
                        table_hbm.at[src_v.at[g + 1]], msg1, gsem).wait()

                    @pl.when(g + 2 < SEG)
                    def _():
                        pltpu.async_copy(
                            table_hbm.at[src_v.at[g + 2]], msg0, gsem)

                    pltpu.sync_copy(msg1, acc_sh.at[dst_v.at[g + 1]], add=True)

            with jax.named_scope("barrier1"):
                plsc.subcore_barrier()
            with jax.named_scope("writeout"):
                pltpu.sync_copy(
                    acc_sh.at[pl.ds(s * rows_per_sub, rows_per_sub)],
                    out_hbm.at[c, pl.ds(s * rows_per_sub, rows_per_sub)])

    return edge_k(table, src3, dst3)


def _tc_scale(degp, x_pad, NP, C):
    BR = 1024

    def body(dp_ref, x_ref, dis_ref, xs_ref):
        p = dp_ref[0] + dp_ref[1]
        deg = 1.0 + jnp.sum(p, axis=1, keepdims=True) * (1.0 / C)
        dis = lax.rsqrt(deg)
        dis_ref[...] = dis
        xs_ref[...] = x_ref[...] * dis

    return pl.pallas_call(
        body,
        grid=(NP // BR,),
        in_specs=[
            pl.BlockSpec((NC, BR, C), lambda i: (0, i, 0)),
            pl.BlockSpec((BR, C), lambda i: (i, 0)),
        ],
        out_specs=[
            pl.BlockSpec((BR, 1), lambda i: (i, 0)),
            pl.BlockSpec((BR, C), lambda i: (i, 0)),
        ],
        out_shape=[
            jax.ShapeDtypeStruct((NP, 1), jnp.float32),
            jax.ShapeDtypeStruct((NP, C), jnp.float32),
        ],
    )(degp, x_pad)


def _tc_mlp(t1a, t1b, xs, dis, W1, b1r, W2, NP, C, CH):
    BR = 1024

    def body(ta, tb, xs_r, dis_r, w1_r, b1_r, w2_r, g_ref):
        d = dis_r[...]
        y1 = d * (ta[...] + tb[...] + xs_r[...])
        h = jnp.dot(y1, w1_r[...], preferred_element_type=jnp.float32,
                    precision=HIGHEST)
        h = jnp.maximum(h + b1_r[...], 0.0)
        g_ref[...] = jnp.dot(d * h, w2_r[...],
                             preferred_element_type=jnp.float32,
                             precision=HIGHEST)

    return pl.pallas_call(
        body,
        grid=(NP // BR,),
        in_specs=[
            pl.BlockSpec((BR, C), lambda i: (i, 0)),
            pl.BlockSpec((BR, C), lambda i: (i, 0)),
            pl.BlockSpec((BR, C), lambda i: (i, 0)),
            pl.BlockSpec((BR, 1), lambda i: (i, 0)),
            pl.BlockSpec((C, CH), lambda i: (0, 0)),
            pl.BlockSpec((1, CH), lambda i: (0, 0)),
            pl.BlockSpec((CH, C), lambda i: (0, 0)),
        ],
        out_specs=pl.BlockSpec((BR, C), lambda i: (i, 0)),
        out_shape=jax.ShapeDtypeStruct((NP, C), jnp.float32),
    )(t1a, t1b, xs, dis, W1, b1r, W2)


def _tc_final(t2a, t2b, g, dis, b2r, NP, C):
    BR = 1024

    def body(ta, tb, g_r, dis_r, b2_r, out_ref):
        out_ref[...] = dis_r[...] * (ta[...] + tb[...] + g_r[...]) + b2_r[...]

    return pl.pallas_call(
        body,
        grid=(NP // BR,),
        in_specs=[
            pl.BlockSpec((BR, C), lambda i: (i, 0)),
            pl.BlockSpec((BR, C), lambda i: (i, 0)),
            pl.BlockSpec((BR, C), lambda i: (i, 0)),
            pl.BlockSpec((BR, 1), lambda i: (i, 0)),
            pl.BlockSpec((1, C), lambda i: (0, 0)),
        ],
        out_specs=pl.BlockSpec((BR, C), lambda i: (i, 0)),
        out_shape=jax.ShapeDtypeStruct((NP, C), jnp.float32),
    )(t2a, t2b, g, dis, b2r)


def kernel(x, edge_index, W1, b1, W2, b2):
    N, C = x.shape
    CH = W1.shape[1]
    E = edge_index.shape[1]

    src = edge_index[0].astype(jnp.int32)
    dst = edge_index[1].astype(jnp.int32)

    NP = -(-N // (NS * CHUNK)) * (NS * CHUNK)
    if NP == N:
        NP += NS * CHUNK
    NSEG = NS * (SF + SS)
    SEG = -(-E // (CHUNK * NSEG))
    SEG += SEG % 2
    EP = NSEG * SEG * CHUNK

    pad_i = jnp.arange(EP - E, dtype=jnp.int32)
    src_p = jnp.concatenate(
        [src, pad_i % N]).reshape(NSEG, SEG, CHUNK)
    dst_p = jnp.concatenate(
        [dst, N + pad_i % (NP - N)]).reshape(NSEG, SEG, CHUNK)
    x_pad = jnp.pad(x, ((0, NP - N), (0, 0)))
    b1r = b1.reshape(1, CH)
    b2r = b2.reshape(1, C)

    ones_tab = jnp.ones((NP, C), jnp.float32)
    degp = _edge_pass(ones_tab, src_p, dst_p, NP, C)
    dis, xs = _tc_scale(degp, x_pad, NP, C)
    t1 = _edge_pass(xs, src_p, dst_p, NP, C)
    g = _tc_mlp(t1[0], t1[1], xs, dis, W1, b1r, W2, NP, C, CH)
    t2 = _edge_pass(g, src_p, dst_p, NP, C)
    out = _tc_final(t2[0], t2[1], g, dis, b2r, NP, C)
    return out[:N]


# speedup vs baseline: 2.8185x; 1.0280x over previous
import functools

import jax
import jax.numpy as jnp
from jax import lax
from jax.experimental import pallas as pl
from jax.experimental.pallas import tpu as pltpu
from jax.experimental.pallas import tpu_sc as plsc

NC = 2
NS = 16
LANES = 16
NW = NC * NS
CHUNK = 128
HIGHEST = lax.Precision.HIGHEST


def _sc_mesh():
    return plsc.VectorSubcoreMesh(core_axis_name="c", subcore_axis_name="s")


def _fill_vmem(ref, rows, cols, value):
    val = jnp.full((LANES,), value, jnp.float32)

    @pl.loop(0, rows)
    def _(i):
        @pl.loop(0, cols, step=LANES)
        def _(j):
            ref[i, pl.ds(j, LANES)] = val


SF = 4
SS = 4


def _edge_pass(table, src3, dst3, NP, C):
    SEG = src3.shape[1]
    rows_per_sub = NP // NS
    spread = rows_per_sub // CHUNK

    @functools.partial(
        pl.kernel,
        out_type=jax.ShapeDtypeStruct((NC, NP, C), jnp.float32),
        mesh=_sc_mesh(),
        scratch_types=[
            pltpu.VMEM((SEG, CHUNK), jnp.int32),
            pltpu.VMEM((SEG, CHUNK), jnp.int32),
            pltpu.VMEM((CHUNK, C), jnp.float32),
            pltpu.VMEM((CHUNK, C), jnp.float32),
            pltpu.VMEM_SHARED((NP, C), jnp.float32),
            pltpu.SemaphoreType.DMA,
        ],
    )
    def edge_k(table_hbm, src_hbm, dst_hbm, out_hbm, src_v, dst_v,
               msg0, msg1, acc_sh, gsem):
        c = lax.axis_index("c")
        s = lax.axis_index("s")
        nseg = jnp.where(c == 0, SF, SS)
        base = jnp.where(c == 0, s * SF, NS * SF + s * SS)

        with jax.named_scope("zero_fill"):
            _fill_vmem(msg0, CHUNK, C, 0.0)

        if True:
            with jax.named_scope("zero_spread"):
                @pl.loop(0, spread)
                def _(j):
                    pltpu.sync_copy(
                        msg0,
                        acc_sh.at[pl.ds(s * rows_per_sub + j * CHUNK, CHUNK)])
            with jax.named_scope("barrier0"):
                plsc.subcore_barrier()

            @pl.loop(0, nseg)
            def _(q):
                with jax.named_scope("idx_load"):
                    pltpu.sync_copy(src_hbm.at[base + q], src_v)
                    pltpu.sync_copy(dst_hbm.at[base + q], dst_v)

                pltpu.async_copy(table_hbm.at[src_v.at[0]], msg0, gsem)

                @pl.loop(0, SEG, step=2)
                @jax.named_scope("chunk_pair")
                def _(g):
                    pltpu.make_async_copy(
                        table_hbm.at[src_v.at[g]], msg0, gsem).wait()
                    pltpu.async_copy(table_hbm.at[src_v.at[g + 1]], msg1, gsem)
                    pltpu.sync_copy(msg0, acc_sh.at[dst_v.at[g]], add=True)
                    pltpu.make_async_copy(
                        table_hbm.at[src_v.at[g + 1]], msg1, gsem).wait()

                    @pl.when(g + 2 < SEG)
                    def _():
                        pltpu.async_copy(
                            table_hbm.at[src_v.at[g + 2]], msg0, gsem)

                    pltpu.sync_copy(msg1, acc_sh.at[dst_v.at[g + 1]], add=True)

            with jax.named_scope("barrier1"):
                plsc.subcore_barrier()
            with jax.named_scope("writeout"):
                pltpu.sync_copy(
                    acc_sh.at[pl.ds(s * rows_per_sub, rows_per_sub)],
                    out_hbm.at[c, pl.ds(s * rows_per_sub, rows_per_sub)])

    return edge_k(table, src3, dst3)


def _tc_scale(degp, x_pad, NP, C):
    BR = 1024

    def body(dp_ref, x_ref, dis_ref, xs_ref):
        p = dp_ref[0] + dp_ref[1]
        deg = 1.0 + jnp.sum(p, axis=1, keepdims=True) * (1.0 / C)
        dis = lax.rsqrt(deg)
        dis_ref[...] = dis
        xs_ref[...] = x_ref[...] * dis

    return pl.pallas_call(
        body,
        grid=(NP // BR,),
        in_specs=[
            pl.BlockSpec((NC, BR, C), lambda i: (0, i, 0)),
            pl.BlockSpec((BR, C), lambda i: (i, 0)),
        ],
        out_specs=[
            pl.BlockSpec((BR, 1), lambda i: (i, 0)),
            pl.BlockSpec((BR, C), lambda i: (i, 0)),
        ],
        out_shape=[
            jax.ShapeDtypeStruct((NP, 1), jnp.float32),
            jax.ShapeDtypeStruct((NP, C), jnp.float32),
        ],
    )(degp, x_pad)


def _tc_mlp(t1a, xs, dis, W1, b1r, W2, NP, C, CH):
    BR = 1024

    def body(t_ref, xs_r, dis_r, w1_r, b1_r, w2_r, g_ref):
        d = dis_r[...]
        y1 = d * (t_ref[0] + t_ref[1] + xs_r[...])
        h = jnp.dot(y1, w1_r[...], preferred_element_type=jnp.float32,
                    precision=HIGHEST)
        h = jnp.maximum(h + b1_r[...], 0.0)
        g_ref[...] = jnp.dot(d * h, w2_r[...],
                             preferred_element_type=jnp.float32,
                             precision=HIGHEST)

    return pl.pallas_call(
        body,
        grid=(NP // BR,),
        in_specs=[
            pl.BlockSpec((NC, BR, C), lambda i: (0, i, 0)),
            pl.BlockSpec((BR, C), lambda i: (i, 0)),
            pl.BlockSpec((BR, 1), lambda i: (i, 0)),
            pl.BlockSpec((C, CH), lambda i: (0, 0)),
            pl.BlockSpec((1, CH), lambda i: (0, 0)),
            pl.BlockSpec((CH, C), lambda i: (0, 0)),
        ],
        out_specs=pl.BlockSpec((BR, C), lambda i: (i, 0)),
        out_shape=jax.ShapeDtypeStruct((NP, C), jnp.float32),
    )(t1a, xs, dis, W1, b1r, W2)


def _tc_final(t2a, g, dis, b2r, NP, C):
    BR = 1024

    def body(t_ref, g_r, dis_r, b2_r, out_ref):
        out_ref[...] = dis_r[...] * (t_ref[0] + t_ref[1] + g_r[...]) + b2_r[...]

    return pl.pallas_call(
        body,
        grid=(NP // BR,),
        in_specs=[
            pl.BlockSpec((NC, BR, C), lambda i: (0, i, 0)),
            pl.BlockSpec((BR, C), lambda i: (i, 0)),
            pl.BlockSpec((BR, 1), lambda i: (i, 0)),
            pl.BlockSpec((1, C), lambda i: (0, 0)),
        ],
        out_specs=pl.BlockSpec((BR, C), lambda i: (i, 0)),
        out_shape=jax.ShapeDtypeStruct((NP, C), jnp.float32),
    )(t2a, g, dis, b2r)


def kernel(x, edge_index, W1, b1, W2, b2):
    N, C = x.shape
    CH = W1.shape[1]
    E = edge_index.shape[1]

    src = edge_index[0].astype(jnp.int32)
    dst = edge_index[1].astype(jnp.int32)

    NP = -(-N // (NS * CHUNK)) * (NS * CHUNK)
    if NP == N:
        NP += NS * CHUNK
    NSEG = NS * (SF + SS)
    SEG = -(-E // (CHUNK * NSEG))
    SEG += SEG % 2
    EP = NSEG * SEG * CHUNK

    pad_i = jnp.arange(EP - E, dtype=jnp.int32)
    src_p = jnp.concatenate(
        [src, pad_i % N]).reshape(NSEG, SEG, CHUNK)
    dst_p = jnp.concatenate(
        [dst, N + pad_i % (NP - N)]).reshape(NSEG, SEG, CHUNK)
    x_pad = jnp.pad(x, ((0, NP - N), (0, 0)))
    b1r = b1.reshape(1, CH)
    b2r = b2.reshape(1, C)

    ones_tab = jnp.ones((NP, C), jnp.float32)
    degp = _edge_pass(ones_tab, src_p, dst_p, NP, C)
    dis, xs = _tc_scale(degp, x_pad, NP, C)
    t1 = _edge_pass(xs, src_p, dst_p, NP, C)
    g = _tc_mlp(t1, xs, dis, W1, b1r, W2, NP, C, CH)
    t2 = _edge_pass(g, src_p, dst_p, NP, C)
    out = _tc_final(t2, g, dis, b2r, NP, C)
    return out[:N]
